# Initial kernel scaffold; baseline (speedup 1.0000x reference)
import jax
import jax.numpy as jnp
from jax.experimental import pallas as pl


def kernel(x, ei0, ei1, ei2, l0_W0, l0_al0, l0_ar0, l0_b0, l0_W1, l0_al1, l0_ar1, l0_b1, l0_W2, l0_al2, l0_ar2, l0_b2, l1_W0, l1_al0, l1_ar0, l1_b0, l1_W1, l1_al1, l1_ar1, l1_b1, l1_W2, l1_al2, l1_ar2, l1_b2, lin_W, lin_b):
    raise NotImplementedError("write your pallas kernel here")


# speedup vs baseline: 34.6355x; 34.6355x over previous
import jax
import jax.numpy as jnp
import numpy as np
from jax import lax
from jax.experimental import pallas as pl
from jax.experimental.pallas import tpu as pltpu
from jax.experimental.pallas import tpu_sc as plsc

_N = 10000
_E = 100000
_HD = 128
_H = 4
_D = 32
_C = 153
_NEG = 0.2
_EPS = 1e-9

_NC, _NS, _L = 2, 16, 16
_NW = _NC * _NS
_CHUNK = 96
_NCH = 33
_EPW = _NCH * _CHUNK
_EPAD = _NW * _EPW
_AW = 144
_NPAD = 10112
_RPT = _NPAD // _NS
_RPC = 79

_BLK = 400
_GRID = _N // _BLK


def _transform(hb, w, av, rv, s1, s2, zo, eo):
    z = jnp.dot(hb, w[...], preferred_element_type=jnp.float32)
    zo[...] = z
    eo[...] = (jnp.dot(z * av[...], s1[...], preferred_element_type=jnp.float32)
               + jnp.dot(z * rv[...], s2[...], preferred_element_type=jnp.float32))


def _tc1_body(h, w0, av0, rv0, w1, av1, rv1, w2, av2, rv2, s1, s2,
              z0, e0, z1, e1, z2, e2):
    hb = h[...]
    for w, av, rv, zo, eo in ((w0, av0, rv0, z0, e0), (w1, av1, rv1, z1, e1),
                              (w2, av2, rv2, z2, e2)):
        _transform(hb, w, av, rv, s1, s2, zo, eo)


def _combine(accs, b0, b1, b2, rsel):
    out = b0[...] + b1[...] + b2[...]
    for a_lo, a_hi in accs:
        nm = a_lo[...] + a_hi[...]
        den = jnp.dot(nm[:, _HD:_AW], rsel[...], preferred_element_type=jnp.float32)
        out = out + nm[:, :_HD] / (den + _EPS)
    return out


def _tc2_body(a00, a01, a10, a11, a20, a21, b0, b1, b2, rsel,
              w0, av0, rv0, w1, av1, rv1, w2, av2, rv2, s1, s2,
              z0, e0, z1, e1, z2, e2):
    hb = jnp.maximum(_combine(((a00, a01), (a10, a11), (a20, a21)),
                              b0, b1, b2, rsel), 0.0)
    for w, av, rv, zo, eo in ((w0, av0, rv0, z0, e0), (w1, av1, rv1, z1, e1),
                              (w2, av2, rv2, z2, e2)):
        _transform(hb, w, av, rv, s1, s2, zo, eo)


def _tc3_body(a00, a01, a10, a11, a20, a21, b0, b1, b2, rsel, lw, lb, out):
    hb = _combine(((a00, a01), (a10, a11), (a20, a21)), b0, b1, b2, rsel)
    out[...] = jnp.dot(hb, lw[...], preferred_element_type=jnp.float32) + lb[...]


_FULL = lambda shape: pl.BlockSpec(shape, lambda i: (0,) * len(shape))
_ROWB = lambda shape: pl.BlockSpec(shape, lambda i: (i,) + (0,) * (len(shape) - 1))

_WSPECS = [_FULL((_HD, _HD)), _FULL((1, _HD)), _FULL((1, _HD))] * 3 + \
          [_FULL((_HD, 16)), _FULL((_HD, 16))]
_ZOUTS = [jax.ShapeDtypeStruct((_N, _HD), jnp.float32),
          jax.ShapeDtypeStruct((_N, 16), jnp.float32)] * 3
_ZSPEC = [_ROWB((_BLK, _HD)), _ROWB((_BLK, 16))] * 3
_ACCSPECS = [_ROWB((_BLK, _AW))] * 6 + [_FULL((1, _HD))] * 3 + [_FULL((16, _HD))]

_tc1 = pl.pallas_call(
    _tc1_body, grid=(_GRID,),
    in_specs=[_ROWB((_BLK, _HD))] + _WSPECS,
    out_specs=_ZSPEC, out_shape=_ZOUTS)

_tc2 = pl.pallas_call(
    _tc2_body, grid=(_GRID,),
    in_specs=_ACCSPECS + _WSPECS,
    out_specs=_ZSPEC, out_shape=_ZOUTS)

_tc3 = pl.pallas_call(
    _tc3_body, grid=(_GRID,),
    in_specs=_ACCSPECS + [_FULL((_HD, 256)), _FULL((1, 256))],
    out_specs=_ROWB((_BLK, 256)),
    out_shape=jax.ShapeDtypeStruct((_N, 256), jnp.float32))


def _sc_body(z_h, eler_h, src_h, dst_h, out_h,
             sidx, didx, zbuf, wzbuf, esb, edb, acc, sz, se1, se2):
    c = lax.axis_index("c")
    s = lax.axis_index("s")
    wid = c * _NS + s

    def _zrow(i, carry):
        for j in range(_AW // _L):
            wzbuf[i, pl.ds(j * _L, _L)] = jnp.zeros((_L,), jnp.float32)
        return carry
    lax.fori_loop(0, _CHUNK, _zrow, 0)
    for k in range(_RPT // _RPC):
        pltpu.sync_copy(wzbuf.at[pl.ds(0, _RPC)],
                        acc.at[pl.ds(s * _RPT + k * _RPC, _RPC)])
    plsc.subcore_barrier()

    def _chunk(j, carry):
        pltpu.sync_copy(src_h.at[wid, j], sidx)
        pltpu.sync_copy(dst_h.at[wid, j], didx)
        cz = pltpu.async_copy(z_h.at[sidx], zbuf, sz)
        c1 = pltpu.async_copy(eler_h.at[sidx], esb, se1)
        c2 = pltpu.async_copy(eler_h.at[didx], edb, se2)
        cz.wait()
        c1.wait()
        c2.wait()
        gbase = wid * _EPW + j * _CHUNK
        for g in range(_CHUNK // _L):
            rows = lax.iota(jnp.int32, _L) + (g * _L)
            live = (gbase + g * _L + lax.iota(jnp.int32, _L)) < _E
            for h in range(_H):
                a = plsc.load_gather(esb, [rows, jnp.full((_L,), h, jnp.int32)])
                b = plsc.load_gather(edb, [rows, jnp.full((_L,), _H + h, jnp.int32)])
                e = a + b
                e = jnp.maximum(e, _NEG * e)
                w = jnp.where(live, jnp.exp(e), 0.0)
                plsc.store_scatter(wzbuf, [rows, jnp.full((_L,), _HD + h, jnp.int32)], w)

        def _scale(i, carry2):
            wrow = wzbuf[i, pl.ds(_HD, _L)]
            for h in range(_H):
                wv = wrow[h]
                for t in range(_D // _L):
                    sl = pl.ds(h * _D + t * _L, _L)
                    wzbuf[i, sl] = zbuf[i, sl] * wv
            return carry2
        lax.fori_loop(0, _CHUNK, _scale, 0)

        pltpu.sync_copy(wzbuf, acc.at[didx], add=True)
        return carry
    lax.fori_loop(0, _NCH, _chunk, 0)

    plsc.subcore_barrier()
    for k in range(_RPT // _RPC):
        r0 = s * _RPT + k * _RPC
        pltpu.sync_copy(acc.at[pl.ds(r0, _RPC)], wzbuf.at[pl.ds(0, _RPC)])
        pltpu.sync_copy(wzbuf.at[pl.ds(0, _RPC)], out_h.at[c, pl.ds(r0, _RPC)])


_sc_conv = pl.kernel(
    _sc_body,
    out_type=jax.ShapeDtypeStruct((_NC, _NPAD, _AW), jnp.float32),
    mesh=plsc.VectorSubcoreMesh(core_axis_name="c", subcore_axis_name="s"),
    compiler_params=pltpu.CompilerParams(use_tc_tiling_on_sc=False,
                                         needs_layout_passes=False),
    scratch_types=[
        pltpu.VMEM((_CHUNK,), jnp.int32),
        pltpu.VMEM((_CHUNK,), jnp.int32),
        pltpu.VMEM((_CHUNK, _HD), jnp.float32),
        pltpu.VMEM((_CHUNK, _AW), jnp.float32),
        pltpu.VMEM((_CHUNK, 16), jnp.float32),
        pltpu.VMEM((_CHUNK, 16), jnp.float32),
        pltpu.VMEM_SHARED((_NPAD, _AW), jnp.float32),
        pltpu.SemaphoreType.DMA,
        pltpu.SemaphoreType.DMA,
        pltpu.SemaphoreType.DMA,
    ])


def _selectors():
    s1 = np.zeros((_HD, 16), np.float32)
    s2 = np.zeros((_HD, 16), np.float32)
    rs = np.zeros((16, _HD), np.float32)
    for h in range(_H):
        s1[h * _D:(h + 1) * _D, h] = 1.0
        s2[h * _D:(h + 1) * _D, _H + h] = 1.0
        rs[h, h * _D:(h + 1) * _D] = 1.0
    return jnp.asarray(s1), jnp.asarray(s2), jnp.asarray(rs)


def _prep_edges(ei):
    pad = jnp.zeros((_EPAD - _E,), jnp.int32)
    src = jnp.concatenate([ei[0], pad]).reshape(_NW, _NCH, _CHUNK)
    dst = jnp.concatenate([ei[1], pad]).reshape(_NW, _NCH, _CHUNK)
    return src, dst


def kernel(x, ei0, ei1, ei2,
           l0_W0, l0_al0, l0_ar0, l0_b0, l0_W1, l0_al1, l0_ar1, l0_b1,
           l0_W2, l0_al2, l0_ar2, l0_b2,
           l1_W0, l1_al0, l1_ar0, l1_b0, l1_W1, l1_al1, l1_ar1, l1_b1,
           l1_W2, l1_al2, l1_ar2, l1_b2, lin_W, lin_b):
    s1, s2, rsel = _selectors()
    edges = [_prep_edges(ei) for ei in (ei0, ei1, ei2)]

    def flat_params(ws, als, ars):
        out = []
        for w, al, ar in zip(ws, als, ars):
            out += [w, al.reshape(1, _HD), ar.reshape(1, _HD)]
        return out

    p0 = flat_params((l0_W0, l0_W1, l0_W2), (l0_al0, l0_al1, l0_al2),
                     (l0_ar0, l0_ar1, l0_ar2))
    p1 = flat_params((l1_W0, l1_W1, l1_W2), (l1_al0, l1_al1, l1_al2),
                     (l1_ar0, l1_ar1, l1_ar2))
    b0 = [b.reshape(1, _HD) for b in (l0_b0, l0_b1, l0_b2)]
    b1 = [b.reshape(1, _HD) for b in (l1_b0, l1_b1, l1_b2)]

    z0, e0, z1, e1, z2, e2 = _tc1(x, *p0, s1, s2)

    accs = []
    for (src, dst), z, e in zip(edges, (z0, z1, z2), (e0, e1, e2)):
        a = _sc_conv(z, e, src, dst)
        accs += [a[0], a[1]]

    z0, e0, z1, e1, z2, e2 = _tc2(*accs, *b0, rsel, *p1, s1, s2)

    accs = []
    for (src, dst), z, e in zip(edges, (z0, z1, z2), (e0, e1, e2)):
        a = _sc_conv(z, e, src, dst)
        accs += [a[0], a[1]]

    lw = jnp.pad(lin_W, ((0, 0), (0, 256 - _C)))
    lb = jnp.pad(lin_b, (0, 256 - _C)).reshape(1, 256)
    out = _tc3(*accs, *b1, rsel, lw, lb)
    return out[:, :_C]


# speedup vs baseline: 42.3697x; 1.2233x over previous
import jax
import jax.numpy as jnp
import numpy as np
from jax import lax
from jax.experimental import pallas as pl
from jax.experimental.pallas import tpu as pltpu
from jax.experimental.pallas import tpu_sc as plsc

_N = 10000
_E = 100000
_HD = 128
_H = 4
_D = 32
_C = 153
_NEG = 0.2
_EPS = 1e-9

_NC, _NS, _L = 2, 16, 16
_NW = _NC * _NS
_CHUNK = 96
_NCH = 34
_EPW = _NCH * _CHUNK
_EPAD = _NW * _EPW
_WW = 16
_NPAD = 10112
_RPT = _NPAD // _NS
_RPC = 79

_BLK = 400
_GRID = _N // _BLK


def _transform(hb, w, av, rv, s1, s2, zo, eo):
    z = jnp.dot(hb, w[...], preferred_element_type=jnp.float32)
    zo[...] = z
    eo[...] = (jnp.dot(z * av[...], s1[...], preferred_element_type=jnp.float32)
               + jnp.dot(z * rv[...], s2[...], preferred_element_type=jnp.float32))


def _tc1_body(h, w0, av0, rv0, w1, av1, rv1, w2, av2, rv2, s1, s2,
              z0, e0, z1, e1, z2, e2):
    hb = h[...]
    for w, av, rv, zo, eo in ((w0, av0, rv0, z0, e0), (w1, av1, rv1, z1, e1),
                              (w2, av2, rv2, z2, e2)):
        _transform(hb, w, av, rv, s1, s2, zo, eo)


def _combine(zaccs, waccs, b0, b1, b2, rsel):
    out = b0[...] + b1[...] + b2[...]
    for (z_lo, z_hi), (w_lo, w_hi) in zip(zaccs, waccs):
        nm = z_lo[...] + z_hi[...]
        den = jnp.dot(w_lo[...] + w_hi[...], rsel[...],
                      preferred_element_type=jnp.float32)
        out = out + nm / (den + _EPS)
    return out


def _tc2_body(a00, a01, a10, a11, a20, a21, q00, q01, q10, q11, q20, q21,
              b0, b1, b2, rsel,
              w0, av0, rv0, w1, av1, rv1, w2, av2, rv2, s1, s2,
              z0, e0, z1, e1, z2, e2):
    hb = jnp.maximum(_combine(((a00, a01), (a10, a11), (a20, a21)),
                              ((q00, q01), (q10, q11), (q20, q21)),
                              b0, b1, b2, rsel), 0.0)
    for w, av, rv, zo, eo in ((w0, av0, rv0, z0, e0), (w1, av1, rv1, z1, e1),
                              (w2, av2, rv2, z2, e2)):
        _transform(hb, w, av, rv, s1, s2, zo, eo)


def _tc3_body(a00, a01, a10, a11, a20, a21, q00, q01, q10, q11, q20, q21,
              b0, b1, b2, rsel, lw, lb, out):
    hb = _combine(((a00, a01), (a10, a11), (a20, a21)),
                  ((q00, q01), (q10, q11), (q20, q21)), b0, b1, b2, rsel)
    out[...] = jnp.dot(hb, lw[...], preferred_element_type=jnp.float32) + lb[...]


_FULL = lambda shape: pl.BlockSpec(shape, lambda i: (0,) * len(shape))
_ROWB = lambda shape: pl.BlockSpec(shape, lambda i: (i,) + (0,) * (len(shape) - 1))

_WSPECS = [_FULL((_HD, _HD)), _FULL((1, _HD)), _FULL((1, _HD))] * 3 + \
          [_FULL((_HD, 16)), _FULL((_HD, 16))]
_ZOUTS = [jax.ShapeDtypeStruct((_N, _HD), jnp.float32),
          jax.ShapeDtypeStruct((_N, 16), jnp.float32)] * 3
_ZSPEC = [_ROWB((_BLK, _HD)), _ROWB((_BLK, 16))] * 3
_ACCSPECS = [_ROWB((_BLK, _HD))] * 6 + [_ROWB((_BLK, _WW))] * 6 + \
            [_FULL((1, _HD))] * 3 + [_FULL((16, _HD))]

_tc1 = pl.pallas_call(
    _tc1_body, grid=(_GRID,),
    in_specs=[_ROWB((_BLK, _HD))] + _WSPECS,
    out_specs=_ZSPEC, out_shape=_ZOUTS)

_tc2 = pl.pallas_call(
    _tc2_body, grid=(_GRID,),
    in_specs=_ACCSPECS + _WSPECS,
    out_specs=_ZSPEC, out_shape=_ZOUTS)

_tc3 = pl.pallas_call(
    _tc3_body, grid=(_GRID,),
    in_specs=_ACCSPECS + [_FULL((_HD, 256)), _FULL((1, 256))],
    out_specs=_ROWB((_BLK, 256)),
    out_shape=jax.ShapeDtypeStruct((_N, 256), jnp.float32))


def _sc_body(z_h, eler_h, src_h, dst_h, oz_h, ow_h,
             sidx0, didx0, zbuf0, esb0, edb0,
             sidx1, didx1, zbuf1, esb1, edb1,
             wbuf, accz, accw, sem0, sem1):
    c = lax.axis_index("c")
    s = lax.axis_index("s")
    wid = c * _NS + s
    sidx = (sidx0, sidx1)
    didx = (didx0, didx1)
    zbuf = (zbuf0, zbuf1)
    esb = (esb0, esb1)
    edb = (edb0, edb1)
    sem = (sem0, sem1)

    def _zrow(i, carry):
        for j in range(_HD // _L):
            zbuf0[i, pl.ds(j * _L, _L)] = jnp.zeros((_L,), jnp.float32)
        wbuf[i, pl.ds(0, _L)] = jnp.zeros((_L,), jnp.float32)
        return carry
    lax.fori_loop(0, _CHUNK, _zrow, 0)
    for k in range(_RPT // _RPC):
        r0 = s * _RPT + k * _RPC
        pltpu.sync_copy(zbuf0.at[pl.ds(0, _RPC)], accz.at[pl.ds(r0, _RPC)])
        pltpu.sync_copy(wbuf.at[pl.ds(0, _RPC)], accw.at[pl.ds(r0, _RPC)])
    plsc.subcore_barrier()

    def _issue(j, b):
        pltpu.sync_copy(src_h.at[wid, j], sidx[b])
        pltpu.sync_copy(dst_h.at[wid, j], didx[b])
        pltpu.async_copy(z_h.at[sidx[b]], zbuf[b], sem[b])
        pltpu.async_copy(eler_h.at[sidx[b]], esb[b], sem[b])
        pltpu.async_copy(eler_h.at[didx[b]], edb[b], sem[b])

    def _process(j, b):
        pltpu.make_async_copy(z_h.at[sidx[b]], zbuf[b], sem[b]).wait()
        pltpu.make_async_copy(eler_h.at[sidx[b]], esb[b], sem[b]).wait()
        pltpu.make_async_copy(eler_h.at[didx[b]], edb[b], sem[b]).wait()
        gbase = wid * _EPW + j * _CHUNK
        for g in range(_CHUNK // _L):
            rows = lax.iota(jnp.int32, _L) + (g * _L)
            live = (gbase + g * _L + lax.iota(jnp.int32, _L)) < _E
            for h in range(_H):
                a = plsc.load_gather(esb[b], [rows, jnp.full((_L,), h, jnp.int32)])
                bb = plsc.load_gather(edb[b], [rows, jnp.full((_L,), _H + h, jnp.int32)])
                e = a + bb
                e = jnp.maximum(e, _NEG * e)
                w = jnp.where(live, jnp.exp(e), 0.0)
                plsc.store_scatter(wbuf, [rows, jnp.full((_L,), h, jnp.int32)], w)

        def _scale(i, carry2):
            wrow = wbuf[i, pl.ds(0, _L)]
            zb = zbuf[b]
            for h in range(_H):
                wv = wrow[h]
                for tt in range(_D // _L):
                    sl = pl.ds(h * _D + tt * _L, _L)
                    zb[i, sl] = zb[i, sl] * wv
            return carry2
        lax.fori_loop(0, _CHUNK, _scale, 0, unroll=4)

        pltpu.sync_copy(zbuf[b], accz.at[didx[b]], add=True)
        pltpu.sync_copy(wbuf, accw.at[didx[b]], add=True)

    _issue(0, 0)

    def _pair(jj, carry):
        j0 = jj * 2
        _issue(j0 + 1, 1)
        _process(j0, 0)

        @pl.when(jj < _NCH // 2 - 1)
        def _():
            _issue(j0 + 2, 0)
        _process(j0 + 1, 1)
        return carry
    lax.fori_loop(0, _NCH // 2, _pair, 0)

    plsc.subcore_barrier()
    for k in range(_RPT // _RPC):
        r0 = s * _RPT + k * _RPC
        pltpu.sync_copy(accz.at[pl.ds(r0, _RPC)], zbuf0.at[pl.ds(0, _RPC)])
        pltpu.sync_copy(zbuf0.at[pl.ds(0, _RPC)], oz_h.at[c, pl.ds(r0, _RPC)])
        pltpu.sync_copy(accw.at[pl.ds(r0, _RPC)], wbuf.at[pl.ds(0, _RPC)])
        pltpu.sync_copy(wbuf.at[pl.ds(0, _RPC)], ow_h.at[c, pl.ds(r0, _RPC)])


_sc_conv = pl.kernel(
    _sc_body,
    out_type=[jax.ShapeDtypeStruct((_NC, _NPAD, _HD), jnp.float32),
              jax.ShapeDtypeStruct((_NC, _NPAD, _WW), jnp.float32)],
    mesh=plsc.VectorSubcoreMesh(core_axis_name="c", subcore_axis_name="s"),
    compiler_params=pltpu.CompilerParams(use_tc_tiling_on_sc=False,
                                         needs_layout_passes=False),
    scratch_types=(
        [pltpu.VMEM((_CHUNK,), jnp.int32),
         pltpu.VMEM((_CHUNK,), jnp.int32),
         pltpu.VMEM((_CHUNK, _HD), jnp.float32),
         pltpu.VMEM((_CHUNK, 16), jnp.float32),
         pltpu.VMEM((_CHUNK, 16), jnp.float32)] * 2
        + [pltpu.VMEM((_CHUNK, _WW), jnp.float32),
           pltpu.VMEM_SHARED((_NPAD, _HD), jnp.float32),
           pltpu.VMEM_SHARED((_NPAD, _WW), jnp.float32),
           pltpu.SemaphoreType.DMA,
           pltpu.SemaphoreType.DMA]))


def _selectors():
    s1 = np.zeros((_HD, 16), np.float32)
    s2 = np.zeros((_HD, 16), np.float32)
    rs = np.zeros((16, _HD), np.float32)
    for h in range(_H):
        s1[h * _D:(h + 1) * _D, h] = 1.0
        s2[h * _D:(h + 1) * _D, _H + h] = 1.0
        rs[h, h * _D:(h + 1) * _D] = 1.0
    return jnp.asarray(s1), jnp.asarray(s2), jnp.asarray(rs)


def _prep_edges(ei):
    pad = jnp.zeros((_EPAD - _E,), jnp.int32)
    src = jnp.concatenate([ei[0], pad]).reshape(_NW, _NCH, _CHUNK)
    dst = jnp.concatenate([ei[1], pad]).reshape(_NW, _NCH, _CHUNK)
    return src, dst


def kernel(x, ei0, ei1, ei2,
           l0_W0, l0_al0, l0_ar0, l0_b0, l0_W1, l0_al1, l0_ar1, l0_b1,
           l0_W2, l0_al2, l0_ar2, l0_b2,
           l1_W0, l1_al0, l1_ar0, l1_b0, l1_W1, l1_al1, l1_ar1, l1_b1,
           l1_W2, l1_al2, l1_ar2, l1_b2, lin_W, lin_b):
    s1, s2, rsel = _selectors()
    edges = [_prep_edges(ei) for ei in (ei0, ei1, ei2)]

    def flat_params(ws, als, ars):
        out = []
        for w, al, ar in zip(ws, als, ars):
            out += [w, al.reshape(1, _HD), ar.reshape(1, _HD)]
        return out

    p0 = flat_params((l0_W0, l0_W1, l0_W2), (l0_al0, l0_al1, l0_al2),
                     (l0_ar0, l0_ar1, l0_ar2))
    p1 = flat_params((l1_W0, l1_W1, l1_W2), (l1_al0, l1_al1, l1_al2),
                     (l1_ar0, l1_ar1, l1_ar2))
    b0 = [b.reshape(1, _HD) for b in (l0_b0, l0_b1, l0_b2)]
    b1 = [b.reshape(1, _HD) for b in (l1_b0, l1_b1, l1_b2)]

    z0, e0, z1, e1, z2, e2 = _tc1(x, *p0, s1, s2)

    zaccs, waccs = [], []
    for (src, dst), z, e in zip(edges, (z0, z1, z2), (e0, e1, e2)):
        oz, ow = _sc_conv(z, e, src, dst)
        zaccs += [oz[0], oz[1]]
        waccs += [ow[0], ow[1]]

    z0, e0, z1, e1, z2, e2 = _tc2(*zaccs, *waccs, *b0, rsel, *p1, s1, s2)

    zaccs, waccs = [], []
    for (src, dst), z, e in zip(edges, (z0, z1, z2), (e0, e1, e2)):
        oz, ow = _sc_conv(z, e, src, dst)
        zaccs += [oz[0], oz[1]]
        waccs += [ow[0], ow[1]]

    lw = jnp.pad(lin_W, ((0, 0), (0, 256 - _C)))
    lb = jnp.pad(lin_b, (0, 256 - _C)).reshape(1, 256)
    out = _tc3(*zaccs, *waccs, *b1, rsel, lw, lb)
    return out[:, :_C]
